# baseline (speedup 1.0000x reference)
import functools

import jax
import jax.numpy as jnp
from jax.experimental import pallas as pl
from jax.experimental.pallas import tpu as pltpu

NSEG = 256
BLK = 8192
HB = BLK // 2
WIN = 64


def _body(n_rows, meta_ref, xlo_ref, xhi_ref, b_ref, w1_ref, b1_ref, w2_ref,
          out_ref, accn, accd):
    i = pl.program_id(0)
    nblk = pl.num_programs(0)

    @pl.when(i == 0)
    def _init():
        accn[...] = jnp.zeros_like(accn)
        accd[...] = jnp.zeros_like(accd)

    xlo = xlo_ref[...].astype(jnp.bfloat16)
    xhi = xhi_ref[...].astype(jnp.bfloat16)
    xcat = jnp.concatenate([xlo, xhi], axis=1)
    h2 = jnp.tanh(
        jnp.dot(xcat, w1_ref[...].astype(jnp.bfloat16),
                preferred_element_type=jnp.float32)
        + b1_ref[...]
    ).astype(jnp.bfloat16)
    w2 = w2_ref[...]
    c = 0.5 * jnp.sum(jnp.abs(w2))
    s2 = jax.lax.dot_general(
        w2.astype(jnp.bfloat16), h2, (((1,), (1,)), ((), ())),
        preferred_element_type=jnp.float32)
    ex2 = jnp.exp(s2 - c).astype(jnp.bfloat16)

    if n_rows % BLK:
        tail = n_rows - (n_rows // BLK) * BLK
        tlo = min(tail, HB)
        thi = max(tail - HB, 0)

        def _mask(args):
            xlo_, xhi_, ex_ = args
            col = jax.lax.broadcasted_iota(jnp.int32, (2, HB), 1)
            rix = jax.lax.broadcasted_iota(jnp.int32, (2, HB), 0)
            thr = jnp.where(rix == 0, tlo, thi)
            ex_ = jnp.where(col < thr, ex_, jnp.bfloat16(0.0))
            row = jax.lax.broadcasted_iota(jnp.int32, (HB, 1), 0)
            xlo_ = jnp.where(row < tlo, xlo_, jnp.bfloat16(0.0))
            xhi_ = jnp.where(row < thi, xhi_, jnp.bfloat16(0.0))
            return xlo_, xhi_, ex_

        xlo, xhi, ex2 = jax.lax.cond(
            i == nblk - 1, _mask, lambda a: a, (xlo, xhi, ex2))

    exb_row = jnp.concatenate([ex2[0:1, :], ex2[1:2, :]], axis=1)
    b_row = b_ref[...].astype(jnp.int16)
    base = pl.multiple_of(meta_ref[2 * i], 8)
    ok = meta_ref[2 * i + 1]
    ones_rhs = jnp.ones((BLK, 128), jnp.bfloat16)

    @pl.when(ok == 1)
    def _windowed():
        rel = b_row - base.astype(jnp.int16)
        ohw = jnp.where(
            jax.lax.broadcasted_iota(jnp.int16, (WIN, BLK), 0) == rel,
            jnp.broadcast_to(exb_row, (WIN, BLK)), jnp.bfloat16(0.0))
        accn[pl.ds(base, WIN), :] += (
            jnp.dot(ohw[:, :HB], xlo, preferred_element_type=jnp.float32)
            + jnp.dot(ohw[:, HB:], xhi, preferred_element_type=jnp.float32))
        accd[pl.ds(base, WIN), :] += jnp.dot(
            ohw, ones_rhs, preferred_element_type=jnp.float32)

    @pl.when(ok == 0)
    def _full():
        ohw = jnp.where(
            jax.lax.broadcasted_iota(jnp.int16, (NSEG, BLK), 0) == b_row,
            jnp.broadcast_to(exb_row, (NSEG, BLK)), jnp.bfloat16(0.0))
        accn[...] += (
            jnp.dot(ohw[:, :HB], xlo, preferred_element_type=jnp.float32)
            + jnp.dot(ohw[:, HB:], xhi, preferred_element_type=jnp.float32))
        accd[...] += jnp.dot(ohw, ones_rhs, preferred_element_type=jnp.float32)

    @pl.when(i == nblk - 1)
    def _fin():
        out_ref[...] = accn[...] / (accd[...] + 1e-16)


@functools.partial(jax.jit, static_argnames=())
def kernel(x, batch, W1, b1, W2, b2):
    n = x.shape[0]
    nblk = (n + BLK - 1) // BLK
    nhb = (n + HB - 1) // HB
    b32 = batch.astype(jnp.int32)
    firsts = b32[jnp.arange(nblk) * BLK]
    lasts = b32[jnp.minimum(jnp.arange(nblk) * BLK + BLK - 1, n - 1)]
    base = jnp.minimum((firsts // 8) * 8, NSEG - WIN)
    ok = (lasts < base + WIN).astype(jnp.int32)
    meta = jnp.stack([base, ok], axis=1).reshape(-1)

    w1bd = jnp.zeros((256, 128), jnp.float32)
    w1bd = w1bd.at[0:128, 0:64].set(W1).at[128:256, 64:128].set(W1)
    w2c = W2.reshape(-1)
    w2bd = jnp.zeros((2, 128), jnp.float32)
    w2bd = w2bd.at[0, 0:64].set(w2c).at[1, 64:128].set(w2c)
    b1bd = jnp.concatenate([b1, b1]).reshape(1, 128)

    grid_spec = pltpu.PrefetchScalarGridSpec(
        num_scalar_prefetch=1,
        grid=(nblk,),
        in_specs=[
            pl.BlockSpec((HB, 128), lambda i, m: (2 * i, 0)),
            pl.BlockSpec((HB, 128),
                         lambda i, m: (jnp.minimum(2 * i + 1, nhb - 1), 0)),
            pl.BlockSpec((1, BLK), lambda i, m: (0, i)),
            pl.BlockSpec((256, 128), lambda i, m: (0, 0)),
            pl.BlockSpec((1, 128), lambda i, m: (0, 0)),
            pl.BlockSpec((2, 128), lambda i, m: (0, 0)),
        ],
        out_specs=pl.BlockSpec((NSEG, 128), lambda i, m: (0, 0)),
        scratch_shapes=[
            pltpu.VMEM((NSEG, 128), jnp.float32),
            pltpu.VMEM((NSEG, 128), jnp.float32),
        ],
    )

    return pl.pallas_call(
        functools.partial(_body, n),
        grid_spec=grid_spec,
        out_shape=jax.ShapeDtypeStruct((NSEG, 128), jnp.float32),
    )(meta, x, x, b32.reshape(1, n), w1bd, b1bd, w2bd)


# speedup vs baseline: 1.1110x; 1.1110x over previous
import functools

import jax
import jax.numpy as jnp
from jax.experimental import pallas as pl
from jax.experimental.pallas import tpu as pltpu

NSEG = 256
BLK = 8192
WIN = 64


def _body(n_rows, meta_ref, x_ref, b_ref, w1_ref, b1_ref, w2_ref,
          out_ref, accn, accd):
    i = pl.program_id(0)
    nblk = pl.num_programs(0)

    @pl.when(i == 0)
    def _init():
        accn[...] = jnp.zeros_like(accn)
        accd[...] = jnp.zeros_like(accd)

    xb = x_ref[...].astype(jnp.bfloat16)
    h = jnp.tanh(
        jnp.dot(xb, w1_ref[...].astype(jnp.bfloat16),
                preferred_element_type=jnp.float32)
        + b1_ref[...]
    ).astype(jnp.bfloat16)
    w2 = w2_ref[...]
    c = jnp.sum(jnp.abs(w2))
    s_row = jax.lax.dot_general(
        w2.astype(jnp.bfloat16), h, (((1,), (1,)), ((), ())),
        preferred_element_type=jnp.float32)
    exb_row = jnp.exp(s_row - c).astype(jnp.bfloat16)

    if n_rows % BLK:
        tail = n_rows - (n_rows // BLK) * BLK

        def _mask(args):
            xb_, ex_ = args
            col = jax.lax.broadcasted_iota(jnp.int32, (1, BLK), 1)
            ex_ = jnp.where(col < tail, ex_, jnp.bfloat16(0.0))
            row = jax.lax.broadcasted_iota(jnp.int32, (BLK, 1), 0)
            xb_ = jnp.where(row < tail, xb_, jnp.bfloat16(0.0))
            return xb_, ex_

        xb, exb_row = jax.lax.cond(
            i == nblk - 1, _mask, lambda a: a, (xb, exb_row))

    b_row = b_ref[...].astype(jnp.int16)
    base = pl.multiple_of(meta_ref[2 * i], 8)
    ok = meta_ref[2 * i + 1]
    ones_rhs = jnp.ones((BLK, 128), jnp.bfloat16)

    @pl.when(ok == 1)
    def _windowed():
        rel = b_row - base.astype(jnp.int16)
        ohw = jnp.where(
            jax.lax.broadcasted_iota(jnp.int16, (WIN, BLK), 0) == rel,
            jnp.broadcast_to(exb_row, (WIN, BLK)), jnp.bfloat16(0.0))
        accn[pl.ds(base, WIN), :] += jnp.dot(
            ohw, xb, preferred_element_type=jnp.float32)
        accd[pl.ds(base, WIN), :] += jnp.dot(
            ohw, ones_rhs, preferred_element_type=jnp.float32)

    @pl.when(ok == 0)
    def _full():
        ohw = jnp.where(
            jax.lax.broadcasted_iota(jnp.int16, (NSEG, BLK), 0) == b_row,
            jnp.broadcast_to(exb_row, (NSEG, BLK)), jnp.bfloat16(0.0))
        accn[...] += jnp.dot(ohw, xb, preferred_element_type=jnp.float32)
        accd[...] += jnp.dot(ohw, ones_rhs, preferred_element_type=jnp.float32)

    @pl.when(i == nblk - 1)
    def _fin():
        out_ref[...] = accn[...] / (accd[...] + 1e-16)


@functools.partial(jax.jit, static_argnames=())
def kernel(x, batch, W1, b1, W2, b2):
    n = x.shape[0]
    nblk = (n + BLK - 1) // BLK
    b32 = batch.astype(jnp.int32)
    firsts = b32[jnp.arange(nblk) * BLK]
    lasts = b32[jnp.minimum(jnp.arange(nblk) * BLK + BLK - 1, n - 1)]
    base = jnp.minimum((firsts // 8) * 8, NSEG - WIN)
    ok = (lasts < base + WIN).astype(jnp.int32)
    meta = jnp.stack([base, ok], axis=1).reshape(-1)

    grid_spec = pltpu.PrefetchScalarGridSpec(
        num_scalar_prefetch=1,
        grid=(nblk,),
        in_specs=[
            pl.BlockSpec((BLK, 128), lambda i, m: (i, 0)),
            pl.BlockSpec((1, BLK), lambda i, m: (0, i)),
            pl.BlockSpec((128, 64), lambda i, m: (0, 0)),
            pl.BlockSpec((1, 64), lambda i, m: (0, 0)),
            pl.BlockSpec((1, 64), lambda i, m: (0, 0)),
        ],
        out_specs=pl.BlockSpec((NSEG, 128), lambda i, m: (0, 0)),
        scratch_shapes=[
            pltpu.VMEM((NSEG, 128), jnp.float32),
            pltpu.VMEM((NSEG, 128), jnp.float32),
        ],
    )

    return pl.pallas_call(
        functools.partial(_body, n),
        grid_spec=grid_spec,
        out_shape=jax.ShapeDtypeStruct((NSEG, 128), jnp.float32),
    )(meta, x, b32.reshape(1, n), W1, b1.reshape(1, -1), W2.reshape(1, -1))
